# baseline (speedup 1.0000x reference)
import functools

import jax
import jax.numpy as jnp
from jax.experimental import pallas as pl
from jax.experimental.pallas import tpu as pltpu


def _decoder_kernel(B, N, HID, s_ref, s2_ref, h0i_ref, w1gh_ref, w1gi_ref,
                    b1g_ref, w1ch_ref, w1ci_ref, b1c_ref, w2gh_ref, w2gi_ref,
                    b2g_ref, w2ch_ref, w2ci_ref, b2c_ref, wpt_ref, bp_ref,
                    out_ref, h0_scr, h1_scr, cur_scr):
    t = pl.program_id(0)

    @pl.when(t == 0)
    def _init():
        h0_scr[...] = h0i_ref[0]
        h1_scr[...] = h0i_ref[1]
        cur_scr[...] = jnp.zeros((8, B * N), jnp.float32)

    def matmul(a, b):
        return jax.lax.dot(a, b, preferred_element_type=jnp.float32)

    def apply_s(x, m_ref):
        return jnp.concatenate(
            [jax.lax.dot_general(
                x[:, b * N:(b + 1) * N], m_ref[...],
                (((1,), (1,)), ((), ())),
                preferred_element_type=jnp.float32) for b in range(B)],
            axis=1)

    def cell(inp, h, wgh_ref, wgi_ref, bg_ref, wch_ref, wci_ref, bc_ref):
        y0 = jnp.concatenate([h, inp], axis=0)
        y1 = apply_s(y0, s_ref)
        y2 = 2.0 * apply_s(y0, s2_ref) - y0
        g = bg_ref[...]
        for k, yk in enumerate((y0, y1, y2)):
            g = (g + matmul(wgh_ref[k], yk[:HID])
                 + matmul(wgi_ref[k], yk[HID:]))
        g = jax.nn.sigmoid(g)
        r, u = g[:HID], g[HID:]
        rh0 = r * h
        rh1 = apply_s(rh0, s_ref)
        rh2 = 2.0 * apply_s(rh0, s2_ref) - rh0
        c = bc_ref[...]
        for k, (rhk, yk) in enumerate(((rh0, y0), (rh1, y1), (rh2, y2))):
            c = c + matmul(wch_ref[k], rhk) + matmul(wci_ref[k], yk[HID:])
        c = jnp.tanh(c)
        return u * h + (1.0 - u) * c

    h0 = cell(cur_scr[...], h0_scr[...], w1gh_ref, w1gi_ref, b1g_ref,
              w1ch_ref, w1ci_ref, b1c_ref)
    h0_scr[...] = h0
    h1 = cell(h0, h1_scr[...], w2gh_ref, w2gi_ref, b2g_ref,
              w2ch_ref, w2ci_ref, b2c_ref)
    h1_scr[...] = h1

    proj = matmul(wpt_ref[...], h1) + bp_ref[...]
    cur_scr[0:1] = proj
    out_ref[0] = proj


def kernel(inputs, initial_hidden_state, supports, W1_gate, b1_gate,
           W1_cand, b1_cand, W2_gate, b2_gate, W2_cand, b2_cand, Wp, bp):
    seq_len, B = inputs.shape[0], inputs.shape[1]
    N = supports.shape[1]
    HID = Wp.shape[0]
    OUT_DIM = Wp.shape[1]
    num_layers = initial_hidden_state.shape[0]
    nm = 3

    S = supports[0]
    S2 = S @ S
    h0i = (initial_hidden_state.reshape(num_layers, B, N, HID)
           .transpose(0, 3, 1, 2).reshape(num_layers, HID, B * N))

    w1g = W1_gate.reshape(1 + HID, nm, 2 * HID)
    w1c = W1_cand.reshape(1 + HID, nm, HID)
    pad = jnp.zeros((7, nm, 2 * HID), jnp.float32)
    padc = jnp.zeros((7, nm, HID), jnp.float32)
    w1gh = w1g[1:].transpose(1, 2, 0)
    w1gi = jnp.concatenate([w1g[:1], pad], 0).transpose(1, 2, 0)
    w1ch = w1c[1:].transpose(1, 2, 0)
    w1ci = jnp.concatenate([w1c[:1], padc], 0).transpose(1, 2, 0)
    w2g = W2_gate.reshape(2 * HID, nm, 2 * HID)
    w2gi = w2g[:HID].transpose(1, 2, 0)
    w2gh = w2g[HID:].transpose(1, 2, 0)
    w2c = W2_cand.reshape(2 * HID, nm, HID)
    w2ci = w2c[:HID].transpose(1, 2, 0)
    w2ch = w2c[HID:].transpose(1, 2, 0)

    b1g = b1_gate.reshape(2 * HID, 1)
    b1c = b1_cand.reshape(HID, 1)
    b2g = b2_gate.reshape(2 * HID, 1)
    b2c = b2_cand.reshape(HID, 1)
    wpt = Wp.T
    bp2 = bp.reshape(1, 1)

    body = functools.partial(_decoder_kernel, B, N, HID)
    full = lambda shape: pl.BlockSpec(shape, lambda t: (0,) * len(shape))
    out = pl.pallas_call(
        body,
        grid=(seq_len,),
        in_specs=[
            full(S.shape), full(S2.shape), full(h0i.shape),
            full(w1gh.shape), full(w1gi.shape), full(b1g.shape),
            full(w1ch.shape), full(w1ci.shape), full(b1c.shape),
            full(w2gh.shape), full(w2gi.shape), full(b2g.shape),
            full(w2ch.shape), full(w2ci.shape), full(b2c.shape),
            full(wpt.shape), full(bp2.shape),
        ],
        out_specs=pl.BlockSpec((1, 1, B * N), lambda t: (t, 0, 0)),
        out_shape=jax.ShapeDtypeStruct((seq_len, 1, B * N), jnp.float32),
        scratch_shapes=[
            pltpu.VMEM((HID, B * N), jnp.float32),
            pltpu.VMEM((HID, B * N), jnp.float32),
            pltpu.VMEM((8, B * N), jnp.float32),
        ],
        compiler_params=pltpu.CompilerParams(
            dimension_semantics=("arbitrary",),
        ),
    )(S, S2, h0i, w1gh, w1gi, b1g, w1ch, w1ci, b1c, w2gh, w2gi, b2g,
      w2ch, w2ci, b2c, wpt, bp2)

    return out.reshape(seq_len, B, N * OUT_DIM)


# speedup vs baseline: 1.3069x; 1.3069x over previous
import functools

import jax
import jax.numpy as jnp
from jax.experimental import pallas as pl
from jax.experimental.pallas import tpu as pltpu


def _decoder_kernel(B, N, HID, s_ref, s2_ref, h0i_ref, w1gh_ref, w1gi_ref,
                    b1g_ref, w1ch_ref, w1ci_ref, b1c_ref, w2gh_ref, w2gi_ref,
                    b2g_ref, w2ch_ref, w2ci_ref, b2c_ref, wpt_ref, bp_ref,
                    out_ref, h0_scr, h1_scr, cur_scr):
    t = pl.program_id(0)

    @pl.when(t == 0)
    def _init():
        h0_scr[...] = h0i_ref[0]
        h1_scr[...] = h0i_ref[1]
        cur_scr[...] = jnp.zeros((8, B * N), jnp.float32)

    def matmul(a, b):
        return jax.lax.dot(a, b, preferred_element_type=jnp.float32)

    def apply_s(x, mt_ref):
        return jnp.concatenate(
            [matmul(x[:, b * N:(b + 1) * N], mt_ref[...]) for b in range(B)],
            axis=1)

    def cell(inp, h, wgh_ref, wgi_ref, bg_ref, wch_ref, wci_ref, bc_ref):
        y0 = jnp.concatenate([h, inp], axis=0)
        y1 = apply_s(y0, s_ref)
        y2 = 2.0 * apply_s(y0, s2_ref) - y0
        g = bg_ref[...]
        for k, yk in enumerate((y0, y1, y2)):
            g = (g + matmul(wgh_ref[k], yk[:HID])
                 + matmul(wgi_ref[k], yk[HID:]))
        g = jax.nn.sigmoid(g)
        r, u = g[:HID], g[HID:]
        rh0 = r * h
        rh1 = apply_s(rh0, s_ref)
        rh2 = 2.0 * apply_s(rh0, s2_ref) - rh0
        c = bc_ref[...]
        for k, (rhk, yk) in enumerate(((rh0, y0), (rh1, y1), (rh2, y2))):
            c = c + matmul(wch_ref[k], rhk) + matmul(wci_ref[k], yk[HID:])
        c = jnp.tanh(c)
        return u * h + (1.0 - u) * c

    h0 = cell(cur_scr[...], h0_scr[...], w1gh_ref, w1gi_ref, b1g_ref,
              w1ch_ref, w1ci_ref, b1c_ref)
    h0_scr[...] = h0
    h1 = cell(h0, h1_scr[...], w2gh_ref, w2gi_ref, b2g_ref,
              w2ch_ref, w2ci_ref, b2c_ref)
    h1_scr[...] = h1

    proj = matmul(wpt_ref[...], h1) + bp_ref[...]
    cur_scr[0:1] = proj
    out_ref[0] = proj


def kernel(inputs, initial_hidden_state, supports, W1_gate, b1_gate,
           W1_cand, b1_cand, W2_gate, b2_gate, W2_cand, b2_cand, Wp, bp):
    seq_len, B = inputs.shape[0], inputs.shape[1]
    N = supports.shape[1]
    HID = Wp.shape[0]
    OUT_DIM = Wp.shape[1]
    num_layers = initial_hidden_state.shape[0]
    nm = 3

    S = supports[0].T
    S2 = S @ S
    h0i = (initial_hidden_state.reshape(num_layers, B, N, HID)
           .transpose(0, 3, 1, 2).reshape(num_layers, HID, B * N))

    w1g = W1_gate.reshape(1 + HID, nm, 2 * HID)
    w1c = W1_cand.reshape(1 + HID, nm, HID)
    pad = jnp.zeros((7, nm, 2 * HID), jnp.float32)
    padc = jnp.zeros((7, nm, HID), jnp.float32)
    w1gh = w1g[1:].transpose(1, 2, 0)
    w1gi = jnp.concatenate([w1g[:1], pad], 0).transpose(1, 2, 0)
    w1ch = w1c[1:].transpose(1, 2, 0)
    w1ci = jnp.concatenate([w1c[:1], padc], 0).transpose(1, 2, 0)
    w2g = W2_gate.reshape(2 * HID, nm, 2 * HID)
    w2gi = w2g[:HID].transpose(1, 2, 0)
    w2gh = w2g[HID:].transpose(1, 2, 0)
    w2c = W2_cand.reshape(2 * HID, nm, HID)
    w2ci = w2c[:HID].transpose(1, 2, 0)
    w2ch = w2c[HID:].transpose(1, 2, 0)

    b1g = b1_gate.reshape(2 * HID, 1)
    b1c = b1_cand.reshape(HID, 1)
    b2g = b2_gate.reshape(2 * HID, 1)
    b2c = b2_cand.reshape(HID, 1)
    wpt = Wp.T
    bp2 = bp.reshape(1, 1)

    body = functools.partial(_decoder_kernel, B, N, HID)
    full = lambda shape: pl.BlockSpec(shape, lambda t: (0,) * len(shape))
    out = pl.pallas_call(
        body,
        grid=(seq_len,),
        in_specs=[
            full(S.shape), full(S2.shape), full(h0i.shape),
            full(w1gh.shape), full(w1gi.shape), full(b1g.shape),
            full(w1ch.shape), full(w1ci.shape), full(b1c.shape),
            full(w2gh.shape), full(w2gi.shape), full(b2g.shape),
            full(w2ch.shape), full(w2ci.shape), full(b2c.shape),
            full(wpt.shape), full(bp2.shape),
        ],
        out_specs=pl.BlockSpec((1, 1, B * N), lambda t: (t, 0, 0)),
        out_shape=jax.ShapeDtypeStruct((seq_len, 1, B * N), jnp.float32),
        scratch_shapes=[
            pltpu.VMEM((HID, B * N), jnp.float32),
            pltpu.VMEM((HID, B * N), jnp.float32),
            pltpu.VMEM((8, B * N), jnp.float32),
        ],
        compiler_params=pltpu.CompilerParams(
            dimension_semantics=("arbitrary",),
        ),
    )(S, S2, h0i, w1gh, w1gi, b1g, w1ch, w1ci, b1c, w2gh, w2gi, b2g,
      w2ch, w2ci, b2c, wpt, bp2)

    return out.reshape(seq_len, B, N * OUT_DIM)


# speedup vs baseline: 1.3354x; 1.0217x over previous
import functools

import jax
import jax.numpy as jnp
from jax.experimental import pallas as pl
from jax.experimental.pallas import tpu as pltpu


def _decoder_kernel(B, N, HID, s_ref, h0i_ref, w1gh_ref, w1gi_ref,
                    b1g_ref, w1ch_ref, w1ci_ref, b1c_ref, w2gh_ref, w2gi_ref,
                    b2g_ref, w2ch_ref, w2ci_ref, b2c_ref, wpt_ref, bp_ref,
                    out_ref, h0_scr, h1_scr, cur_scr):
    t = pl.program_id(0)

    @pl.when(t == 0)
    def _init():
        h0_scr[...] = h0i_ref[0]
        h1_scr[...] = h0i_ref[1]
        cur_scr[...] = jnp.zeros((8, B * N), jnp.float32)

    def matmul(a, b):
        return jax.lax.dot(a, b, preferred_element_type=jnp.float32)

    def apply_s(x, mt_ref):
        return jnp.concatenate(
            [matmul(x[:, b * N:(b + 1) * N], mt_ref[...]) for b in range(B)],
            axis=1)

    def cell(inp, h, wgh_ref, wgi_ref, bg_ref, wch_ref, wci_ref, bc_ref):
        y0 = jnp.concatenate([h, inp], axis=0)
        y1 = apply_s(y0, s_ref)
        y2 = 2.0 * apply_s(y1, s_ref) - y0
        g = bg_ref[...]
        for k, yk in enumerate((y0, y1, y2)):
            g = (g + matmul(wgh_ref[k], yk[:HID])
                 + matmul(wgi_ref[k], yk[HID:]))
        g = jax.nn.sigmoid(g)
        r, u = g[:HID], g[HID:]
        rh0 = r * h
        rh1 = apply_s(rh0, s_ref)
        rh2 = 2.0 * apply_s(rh1, s_ref) - rh0
        c = bc_ref[...]
        for k, (rhk, yk) in enumerate(((rh0, y0), (rh1, y1), (rh2, y2))):
            c = c + matmul(wch_ref[k], rhk) + matmul(wci_ref[k], yk[HID:])
        c = jnp.tanh(c)
        return u * h + (1.0 - u) * c

    h0 = cell(cur_scr[...], h0_scr[...], w1gh_ref, w1gi_ref, b1g_ref,
              w1ch_ref, w1ci_ref, b1c_ref)
    h0_scr[...] = h0
    h1 = cell(h0, h1_scr[...], w2gh_ref, w2gi_ref, b2g_ref,
              w2ch_ref, w2ci_ref, b2c_ref)
    h1_scr[...] = h1

    proj = matmul(wpt_ref[...], h1) + bp_ref[...]
    cur_scr[0:1] = proj
    out_ref[0] = proj


def kernel(inputs, initial_hidden_state, supports, W1_gate, b1_gate,
           W1_cand, b1_cand, W2_gate, b2_gate, W2_cand, b2_cand, Wp, bp):
    seq_len, B = inputs.shape[0], inputs.shape[1]
    N = supports.shape[1]
    HID = Wp.shape[0]
    OUT_DIM = Wp.shape[1]
    num_layers = initial_hidden_state.shape[0]
    nm = 3

    S = supports[0].T
    h0i = (initial_hidden_state.reshape(num_layers, B, N, HID)
           .transpose(0, 3, 1, 2).reshape(num_layers, HID, B * N))

    w1g = W1_gate.reshape(1 + HID, nm, 2 * HID)
    w1c = W1_cand.reshape(1 + HID, nm, HID)
    pad = jnp.zeros((7, nm, 2 * HID), jnp.float32)
    padc = jnp.zeros((7, nm, HID), jnp.float32)
    w1gh = w1g[1:].transpose(1, 2, 0)
    w1gi = jnp.concatenate([w1g[:1], pad], 0).transpose(1, 2, 0)
    w1ch = w1c[1:].transpose(1, 2, 0)
    w1ci = jnp.concatenate([w1c[:1], padc], 0).transpose(1, 2, 0)
    w2g = W2_gate.reshape(2 * HID, nm, 2 * HID)
    w2gi = w2g[:HID].transpose(1, 2, 0)
    w2gh = w2g[HID:].transpose(1, 2, 0)
    w2c = W2_cand.reshape(2 * HID, nm, HID)
    w2ci = w2c[:HID].transpose(1, 2, 0)
    w2ch = w2c[HID:].transpose(1, 2, 0)

    b1g = b1_gate.reshape(2 * HID, 1)
    b1c = b1_cand.reshape(HID, 1)
    b2g = b2_gate.reshape(2 * HID, 1)
    b2c = b2_cand.reshape(HID, 1)
    wpt = Wp.T
    bp2 = bp.reshape(1, 1)

    body = functools.partial(_decoder_kernel, B, N, HID)
    full = lambda shape: pl.BlockSpec(shape, lambda t: (0,) * len(shape))
    out = pl.pallas_call(
        body,
        grid=(seq_len,),
        in_specs=[
            full(S.shape), full(h0i.shape),
            full(w1gh.shape), full(w1gi.shape), full(b1g.shape),
            full(w1ch.shape), full(w1ci.shape), full(b1c.shape),
            full(w2gh.shape), full(w2gi.shape), full(b2g.shape),
            full(w2ch.shape), full(w2ci.shape), full(b2c.shape),
            full(wpt.shape), full(bp2.shape),
        ],
        out_specs=pl.BlockSpec((1, 1, B * N), lambda t: (t, 0, 0)),
        out_shape=jax.ShapeDtypeStruct((seq_len, 1, B * N), jnp.float32),
        scratch_shapes=[
            pltpu.VMEM((HID, B * N), jnp.float32),
            pltpu.VMEM((HID, B * N), jnp.float32),
            pltpu.VMEM((8, B * N), jnp.float32),
        ],
        compiler_params=pltpu.CompilerParams(
            dimension_semantics=("arbitrary",),
        ),
    )(S, h0i, w1gh, w1gi, b1g, w1ch, w1ci, b1c, w2gh, w2gi, b2g,
      w2ch, w2ci, b2c, wpt, bp2)

    return out.reshape(seq_len, B, N * OUT_DIM)


# speedup vs baseline: 1.7523x; 1.3123x over previous
import functools

import jax
import jax.numpy as jnp
from jax.experimental import pallas as pl
from jax.experimental.pallas import tpu as pltpu


def _decoder_kernel(B, N, HID, s_ref, h0i_ref, w1g_ref, b1g_ref,
                    w1c_ref, b1c_ref, w2g_ref, b2g_ref, w2c_ref, b2c_ref,
                    wpt_ref, bp_ref, out_ref, h0_scr, h1_scr, cur_scr):
    t = pl.program_id(0)

    @pl.when(t == 0)
    def _init():
        h0_scr[...] = h0i_ref[0]
        h1_scr[...] = h0i_ref[1]
        cur_scr[...] = jnp.zeros((8, B * N), jnp.float32)

    def matmul(a, b):
        return jax.lax.dot(a, b, preferred_element_type=jnp.float32)

    def apply_s(x, mt_ref):
        return jnp.concatenate(
            [matmul(x[:, b * N:(b + 1) * N], mt_ref[...]) for b in range(B)],
            axis=1)

    def cell(inp, h, wg_ref, bg_ref, wc_ref, bc_ref):
        y0 = jnp.concatenate([h, inp], axis=0)
        y1 = apply_s(y0, s_ref)
        y2 = 2.0 * apply_s(y1, s_ref) - y0
        ycat = jnp.concatenate([y0, y1, y2], axis=0)
        g = jax.nn.sigmoid(matmul(wg_ref[...], ycat) + bg_ref[...])
        r, u = g[:HID], g[HID:]
        z0 = jnp.concatenate([r * h, inp], axis=0)
        z1 = apply_s(z0, s_ref)
        z2 = 2.0 * apply_s(z1, s_ref) - z0
        zcat = jnp.concatenate([z0, z1, z2], axis=0)
        c = jnp.tanh(matmul(wc_ref[...], zcat) + bc_ref[...])
        return u * h + (1.0 - u) * c

    h0 = cell(cur_scr[...], h0_scr[...], w1g_ref, b1g_ref, w1c_ref, b1c_ref)
    h0_scr[...] = h0
    h1 = cell(h0, h1_scr[...], w2g_ref, b2g_ref, w2c_ref, b2c_ref)
    h1_scr[...] = h1

    proj = matmul(wpt_ref[...], h1) + bp_ref[...]
    cur_scr[0:1] = proj
    out_ref[0] = proj


def kernel(inputs, initial_hidden_state, supports, W1_gate, b1_gate,
           W1_cand, b1_cand, W2_gate, b2_gate, W2_cand, b2_cand, Wp, bp):
    seq_len, B = inputs.shape[0], inputs.shape[1]
    N = supports.shape[1]
    HID = Wp.shape[0]
    OUT_DIM = Wp.shape[1]
    num_layers = initial_hidden_state.shape[0]
    nm = 3

    S = supports[0].T
    h0i = (initial_hidden_state.reshape(num_layers, B, N, HID)
           .transpose(0, 3, 1, 2).reshape(num_layers, HID, B * N))

    def prep_w(W, in_rows, out_cols, h_lo, h_hi, pad_rows):
        w = W.reshape(in_rows, nm, out_cols)
        hpart = w[h_lo:h_hi]
        ipart = jnp.concatenate(
            [w[:h_lo], w[h_hi:],
             jnp.zeros((pad_rows, nm, out_cols), jnp.float32)], axis=0)
        blk = jnp.concatenate([hpart, ipart], axis=0)
        return blk.transpose(1, 0, 2).reshape(-1, out_cols).T

    w1gcat = prep_w(W1_gate, 1 + HID, 2 * HID, 1, 1 + HID, 7)
    w1ccat = prep_w(W1_cand, 1 + HID, HID, 1, 1 + HID, 7)
    w2gcat = prep_w(W2_gate, 2 * HID, 2 * HID, HID, 2 * HID, 0)
    w2ccat = prep_w(W2_cand, 2 * HID, HID, HID, 2 * HID, 0)

    b1g = b1_gate.reshape(2 * HID, 1)
    b1c = b1_cand.reshape(HID, 1)
    b2g = b2_gate.reshape(2 * HID, 1)
    b2c = b2_cand.reshape(HID, 1)
    wpt = Wp.T
    bp2 = bp.reshape(1, 1)

    body = functools.partial(_decoder_kernel, B, N, HID)
    full = lambda shape: pl.BlockSpec(shape, lambda t: (0,) * len(shape))
    out = pl.pallas_call(
        body,
        grid=(seq_len,),
        in_specs=[
            full(S.shape), full(h0i.shape),
            full(w1gcat.shape), full(b1g.shape),
            full(w1ccat.shape), full(b1c.shape),
            full(w2gcat.shape), full(b2g.shape),
            full(w2ccat.shape), full(b2c.shape),
            full(wpt.shape), full(bp2.shape),
        ],
        out_specs=pl.BlockSpec((1, 1, B * N), lambda t: (t, 0, 0)),
        out_shape=jax.ShapeDtypeStruct((seq_len, 1, B * N), jnp.float32),
        scratch_shapes=[
            pltpu.VMEM((HID, B * N), jnp.float32),
            pltpu.VMEM((HID, B * N), jnp.float32),
            pltpu.VMEM((8, B * N), jnp.float32),
        ],
        compiler_params=pltpu.CompilerParams(
            dimension_semantics=("arbitrary",),
        ),
    )(S, h0i, w1gcat, b1g, w1ccat, b1c, w2gcat, b2g, w2ccat, b2c, wpt, bp2)

    return out.reshape(seq_len, B, N * OUT_DIM)


# speedup vs baseline: 1.7652x; 1.0073x over previous
import functools

import jax
import jax.numpy as jnp
from jax.experimental import pallas as pl
from jax.experimental.pallas import tpu as pltpu


def _decoder_kernel(B, N, HID, s_ref, h0i_ref, w1g_ref, b1g_ref,
                    w1c_ref, b1c_ref, w2g_ref, b2g_ref, w2c_ref, b2c_ref,
                    wpt_ref, bp_ref, out_ref, h0_scr, h1_scr, cur_scr,
                    y_scr, z_scr):
    t = pl.program_id(0)

    @pl.when(t == 0)
    def _init():
        h0_scr[...] = h0i_ref[0]
        h1_scr[...] = h0i_ref[1]
        cur_scr[...] = jnp.zeros((8, B * N), jnp.float32)

    def matmul(a, b):
        return jax.lax.dot(a, b, preferred_element_type=jnp.float32)

    def apply_s(scr, lo, F):
        return jnp.concatenate(
            [matmul(scr[lo:lo + F, b * N:(b + 1) * N], s_ref[...])
             for b in range(B)], axis=1)

    def cell(inp, h, F, wg_ref, bg_ref, wc_ref, bc_ref, scr_y, scr_z):
        y0 = jnp.concatenate([h, inp], axis=0)
        scr_y[0:F] = y0.astype(jnp.bfloat16)
        y1 = apply_s(scr_y, 0, F)
        scr_y[F:2 * F] = y1.astype(jnp.bfloat16)
        y2 = 2.0 * apply_s(scr_y, F, F) - y0
        scr_y[2 * F:3 * F] = y2.astype(jnp.bfloat16)
        g = jax.nn.sigmoid(matmul(wg_ref[...], scr_y[0:3 * F])
                           + bg_ref[...])
        r, u = g[:HID], g[HID:]
        z0 = jnp.concatenate([r * h, inp], axis=0)
        scr_z[0:F] = z0.astype(jnp.bfloat16)
        z1 = apply_s(scr_z, 0, F)
        scr_z[F:2 * F] = z1.astype(jnp.bfloat16)
        z2 = 2.0 * apply_s(scr_z, F, F) - z0
        scr_z[2 * F:3 * F] = z2.astype(jnp.bfloat16)
        c = jnp.tanh(matmul(wc_ref[...], scr_z[0:3 * F]) + bc_ref[...])
        return u * h + (1.0 - u) * c

    h0 = cell(cur_scr[...], h0_scr[...], HID + 8, w1g_ref, b1g_ref,
              w1c_ref, b1c_ref, y_scr, z_scr)
    h0_scr[...] = h0
    h1 = cell(h0, h1_scr[...], 2 * HID, w2g_ref, b2g_ref,
              w2c_ref, b2c_ref, y_scr, z_scr)
    h1_scr[...] = h1

    proj = matmul(wpt_ref[...], h1) + bp_ref[...]
    cur_scr[0:1] = proj
    out_ref[0] = proj


def kernel(inputs, initial_hidden_state, supports, W1_gate, b1_gate,
           W1_cand, b1_cand, W2_gate, b2_gate, W2_cand, b2_cand, Wp, bp):
    seq_len, B = inputs.shape[0], inputs.shape[1]
    N = supports.shape[1]
    HID = Wp.shape[0]
    OUT_DIM = Wp.shape[1]
    num_layers = initial_hidden_state.shape[0]
    nm = 3

    S = supports[0].T.astype(jnp.bfloat16)
    h0i = (initial_hidden_state.reshape(num_layers, B, N, HID)
           .transpose(0, 3, 1, 2).reshape(num_layers, HID, B * N))

    def prep_w(W, in_rows, out_cols, h_lo, h_hi, pad_rows):
        w = W.reshape(in_rows, nm, out_cols)
        hpart = w[h_lo:h_hi]
        ipart = jnp.concatenate(
            [w[:h_lo], w[h_hi:],
             jnp.zeros((pad_rows, nm, out_cols), jnp.float32)], axis=0)
        blk = jnp.concatenate([hpart, ipart], axis=0)
        wcat = blk.transpose(1, 0, 2).reshape(-1, out_cols).T
        return wcat.astype(jnp.bfloat16)

    w1gcat = prep_w(W1_gate, 1 + HID, 2 * HID, 1, 1 + HID, 7)
    w1ccat = prep_w(W1_cand, 1 + HID, HID, 1, 1 + HID, 7)
    w2gcat = prep_w(W2_gate, 2 * HID, 2 * HID, HID, 2 * HID, 0)
    w2ccat = prep_w(W2_cand, 2 * HID, HID, HID, 2 * HID, 0)

    b1g = b1_gate.reshape(2 * HID, 1)
    b1c = b1_cand.reshape(HID, 1)
    b2g = b2_gate.reshape(2 * HID, 1)
    b2c = b2_cand.reshape(HID, 1)
    wpt = Wp.T
    bp2 = bp.reshape(1, 1)

    body = functools.partial(_decoder_kernel, B, N, HID)
    full = lambda shape: pl.BlockSpec(shape, lambda t: (0,) * len(shape))
    out = pl.pallas_call(
        body,
        grid=(seq_len,),
        in_specs=[
            full(S.shape), full(h0i.shape),
            full(w1gcat.shape), full(b1g.shape),
            full(w1ccat.shape), full(b1c.shape),
            full(w2gcat.shape), full(b2g.shape),
            full(w2ccat.shape), full(b2c.shape),
            full(wpt.shape), full(bp2.shape),
        ],
        out_specs=pl.BlockSpec((1, 1, B * N), lambda t: (t, 0, 0)),
        out_shape=jax.ShapeDtypeStruct((seq_len, 1, B * N), jnp.float32),
        scratch_shapes=[
            pltpu.VMEM((HID, B * N), jnp.float32),
            pltpu.VMEM((HID, B * N), jnp.float32),
            pltpu.VMEM((8, B * N), jnp.float32),
            pltpu.VMEM((3 * 2 * HID, B * N), jnp.bfloat16),
            pltpu.VMEM((3 * 2 * HID, B * N), jnp.bfloat16),
        ],
        compiler_params=pltpu.CompilerParams(
            dimension_semantics=("arbitrary",),
        ),
    )(S, h0i, w1gcat, b1g, w1ccat, b1c, w2gcat, b2g, w2ccat, b2c, wpt, bp2)

    return out.reshape(seq_len, B, N * OUT_DIM)
